# baseline (speedup 1.0000x reference)
import functools

import jax
import jax.numpy as jnp
from jax import lax
from jax.experimental import pallas as pl
from jax.experimental.pallas import tpu as pltpu
from jax.experimental.pallas import tpu_sc as plsc

NC = 2
NS = 16
L = 16
NW = NC * NS

M = 65536
B = 16384
NOPE = 512
ROPE = 64
D = NOPE + ROPE

R = M // NW
CP = 16
CH = 64
NCHMAX = R // CH

_SENT = 0x7FFFFFFF

_GATHER_DNUMS = lax.GatherDimensionNumbers(
    offset_dims=(), collapsed_slice_dims=(0,), start_index_map=(0,))


def _lane_shift_up(x, iota):
    idx = jnp.minimum(iota + 1, L - 1)
    return lax.gather(x, idx[:, None], _GATHER_DNUMS, slice_sizes=(1,),
                      mode=lax.GatherScatterMode.PROMISE_IN_BOUNDS)


def _sc_body(kv_hbm, loc_hbm, nope_hbm, rope_hbm, out_hbm,
             loc_v, table_v, mlist_v, wlist_v, cbufa_v, cbufb_v,
             cbufc_v, cbufd_v, pbufa_v, pbufb_v, pbufc_v,
             sem_a, sem_b, sem_c, sem_d, sem_r):
    wid = lax.axis_index("s") * NC + lax.axis_index("c")
    r0 = wid * R

    def copy_grp(p, carry):
        base = r0 + p * 4 * CP
        g0 = pltpu.async_copy(kv_hbm.at[pl.ds(base, CP)], cbufa_v, sem_a)
        g1 = pltpu.async_copy(kv_hbm.at[pl.ds(base + CP, CP)], cbufb_v, sem_b)
        g2 = pltpu.async_copy(kv_hbm.at[pl.ds(base + 2 * CP, CP)], cbufc_v,
                              sem_c)
        g3 = pltpu.async_copy(kv_hbm.at[pl.ds(base + 3 * CP, CP)], cbufd_v,
                              sem_d)
        g0.wait()
        s0 = pltpu.async_copy(cbufa_v, out_hbm.at[pl.ds(base, CP)], sem_a)
        g1.wait()
        s1 = pltpu.async_copy(cbufb_v, out_hbm.at[pl.ds(base + CP, CP)], sem_b)
        g2.wait()
        s2 = pltpu.async_copy(cbufc_v, out_hbm.at[pl.ds(base + 2 * CP, CP)],
                              sem_c)
        g3.wait()
        s3 = pltpu.async_copy(cbufd_v, out_hbm.at[pl.ds(base + 3 * CP, CP)],
                              sem_d)
        s0.wait()
        s1.wait()
        s2.wait()
        s3.wait()
        return carry

    lax.fori_loop(0, R // (4 * CP), copy_grp, 0)

    pltpu.sync_copy(loc_hbm, loc_v)

    iota = lax.iota(jnp.int32, L)

    neg1 = jnp.full((L,), -1, jnp.int32)

    def init_body(i, carry):
        table_v[pl.ds(i * L, L)] = neg1
        return carry

    lax.fori_loop(0, R // L, init_body, 0)

    def scan_body(c, carry):
        idx = loc_v[pl.ds(c * L, L)]
        ival = c * L + iota
        rel = idx - r0
        in_range = (rel >= 0) & (rel < R)
        comp = jnp.where(in_range, (idx << 14) | ival, jnp.int32(_SENT))
        comp_s, _ = plsc.sort_key_val(comp, comp)
        valid = comp_s != jnp.int32(_SENT)
        idx_s = lax.shift_right_arithmetic(comp_s, 14)
        ival_s = comp_s & jnp.int32(16383)
        nxt = _lane_shift_up(idx_s, iota)
        is_last = valid & ((nxt != idx_s) | (iota == L - 1))
        rel_s = jnp.where(valid, idx_s - r0, 0)
        plsc.store_scatter(table_v, [rel_s], ival_s, mask=is_last)
        return carry

    lax.fori_loop(0, B // L, scan_body, 0)

    def comp_body(v, cnt_vec):
        w = table_v[pl.ds(v * L, L)]
        mask = w >= 0
        m_vec = r0 + v * L + iota
        inc = jnp.where(mask, jnp.int32(1), jnp.int32(0))
        pos = cnt_vec + plsc.cumsum(inc) - 1
        row = lax.shift_right_logical(pos, 6)
        col = pos & jnp.int32(CH - 1)
        plsc.store_scatter(mlist_v, [row, col], m_vec, mask=mask)
        plsc.store_scatter(wlist_v, [row, col], w, mask=mask)
        return cnt_vec + plsc.all_reduce_population_count(mask)

    cnt_vec = lax.fori_loop(0, R // L, comp_body, jnp.zeros((L,), jnp.int32))
    cnt = cnt_vec[0]
    nch = (cnt + CH - 1) // CH
    pad_end = nch * CH

    m0 = mlist_v[0, pl.ds(0, L)][0]
    w0 = wlist_v[0, pl.ds(0, L)][0]

    def pad_body(p, carry):
        pos = cnt + p * L + iota
        maskp = pos < pad_end
        row = lax.shift_right_logical(pos, 6)
        col = pos & jnp.int32(CH - 1)
        plsc.store_scatter(mlist_v, [row, col], jnp.full((L,), 1, jnp.int32) * m0,
                           mask=maskp)
        plsc.store_scatter(wlist_v, [row, col], jnp.full((L,), 1, jnp.int32) * w0,
                           mask=maskp)
        return carry

    lax.fori_loop(0, CH // L, pad_body, 0)

    def chunk_body(k, carry):
        widx = wlist_v.at[k]
        midx = mlist_v.at[k]
        g0 = pltpu.async_copy(
            nope_hbm.at[:, 0, pl.ds(0, 128)].at[widx], pbufa_v, sem_a)
        gr = pltpu.async_copy(rope_hbm.at[:, 0, pl.ds(0, 128)].at[widx],
                              pbufc_v, sem_c)
        g0.wait()
        for j in range(1, 4):
            buf_prev = pbufa_v if (j - 1) % 2 == 0 else pbufb_v
            buf_cur = pbufb_v if (j - 1) % 2 == 0 else pbufa_v
            g = pltpu.async_copy(
                nope_hbm.at[:, 0, pl.ds(128 * j, 128)].at[widx], buf_cur, sem_b)
            s = pltpu.async_copy(
                buf_prev, out_hbm.at[:, 0, pl.ds(128 * (j - 1), 128)].at[midx],
                sem_a)
            g.wait()
            s.wait()
        buf_last = pbufb_v if 3 % 2 == 1 else pbufa_v
        s3 = pltpu.async_copy(
            buf_last, out_hbm.at[:, 0, pl.ds(128 * 3, 128)].at[midx], sem_b)
        mvecs = [mlist_v[k, pl.ds(g * L, L)] for g in range(CH // L)]
        s3.wait()
        gr.wait()
        cbufs = [cbufa_v, cbufb_v, cbufc_v, cbufd_v]
        for i in range(CH):
            half = cbufs[i // CP]
            for c in range(ROPE // L):
                half[i % CP, 0, pl.ds(NOPE + c * L, L)] = \
                    pbufc_v[i, pl.ds(c * L, L)]
        descs = []
        for i in range(CH):
            half = cbufs[i // CP]
            m = mvecs[i // L][i % L]
            descs.append(
                pltpu.async_copy(half.at[pl.ds(i % CP, 1), 0, pl.ds(NOPE, ROPE)],
                                 out_hbm.at[pl.ds(m, 1), 0, pl.ds(NOPE, ROPE)],
                                 sem_r))
        for desc in descs:
            desc.wait()
        return carry

    lax.fori_loop(0, nch, chunk_body, 0)


@functools.partial(
    pl.kernel,
    out_type=jax.ShapeDtypeStruct((M, 1, D), jnp.float32),
    mesh=plsc.VectorSubcoreMesh(core_axis_name="c", subcore_axis_name="s"),
    compiler_params=pltpu.CompilerParams(
        needs_layout_passes=False, use_tc_tiling_on_sc=True),
    scratch_types=[
        pltpu.VMEM((B,), jnp.int32),
        pltpu.VMEM((R,), jnp.int32),
        pltpu.VMEM((NCHMAX, CH), jnp.int32),
        pltpu.VMEM((NCHMAX, CH), jnp.int32),
        pltpu.VMEM((CP, 1, D), jnp.float32),
        pltpu.VMEM((CP, 1, D), jnp.float32),
        pltpu.VMEM((CP, 1, D), jnp.float32),
        pltpu.VMEM((CP, 1, D), jnp.float32),
        pltpu.VMEM((CH, 128), jnp.float32),
        pltpu.VMEM((CH, 128), jnp.float32),
        pltpu.VMEM((CH, 128), jnp.float32),
        pltpu.SemaphoreType.DMA,
        pltpu.SemaphoreType.DMA,
        pltpu.SemaphoreType.DMA,
        pltpu.SemaphoreType.DMA,
        pltpu.SemaphoreType.DMA,
    ],
)
def _sc_scatter(kv_hbm, loc_hbm, nope_hbm, rope_hbm, out_hbm, *rest):
    _sc_body(kv_hbm, loc_hbm, nope_hbm, rope_hbm, out_hbm, *rest)


def kernel(kv_buffer, loc, cache_k_nope, cache_k_rope):
    loc32 = loc.astype(jnp.int32)
    rope_p = jnp.concatenate(
        [cache_k_rope, jnp.zeros((B, 1, 128 - ROPE), jnp.float32)], axis=-1)
    return _sc_scatter(kv_buffer, loc32, cache_k_nope, rope_p)


# speedup vs baseline: 1.1497x; 1.1497x over previous
import functools

import jax
import jax.numpy as jnp
from jax import lax
from jax.experimental import pallas as pl
from jax.experimental.pallas import tpu as pltpu
from jax.experimental.pallas import tpu_sc as plsc

NC = 2
NS = 16
L = 16
NW = NC * NS

M = 65536
B = 16384
NOPE = 512
ROPE = 64
D = NOPE + ROPE

R = M // NW
CP = 16
CH = 64
NCHMAX = R // CH

_SENT = 0x7FFFFFFF

_GATHER_DNUMS = lax.GatherDimensionNumbers(
    offset_dims=(), collapsed_slice_dims=(0,), start_index_map=(0,))


def _lane_shift_up(x, iota):
    idx = jnp.minimum(iota + 1, L - 1)
    return lax.gather(x, idx[:, None], _GATHER_DNUMS, slice_sizes=(1,),
                      mode=lax.GatherScatterMode.PROMISE_IN_BOUNDS)


def _sc_body(loc_hbm, nope_hbm, rope_hbm, out_hbm,
             loc_v, table_v, mlist_v, wlist_v, cbufa_v, cbufb_v,
             cbufc_v, cbufd_v, pbufa_v, pbufb_v, pbufc_v,
             sem_a, sem_b, sem_c, sem_d, sem_r):
    wid = lax.axis_index("s") * NC + lax.axis_index("c")
    r0 = wid * R

    pltpu.sync_copy(loc_hbm, loc_v)

    iota = lax.iota(jnp.int32, L)

    neg1 = jnp.full((L,), -1, jnp.int32)

    def init_body(i, carry):
        table_v[pl.ds(i * L, L)] = neg1
        return carry

    lax.fori_loop(0, R // L, init_body, 0)

    def scan_body(c, carry):
        idx = loc_v[pl.ds(c * L, L)]
        ival = c * L + iota
        rel = idx - r0
        in_range = (rel >= 0) & (rel < R)
        comp = jnp.where(in_range, (idx << 14) | ival, jnp.int32(_SENT))
        comp_s, _ = plsc.sort_key_val(comp, comp)
        valid = comp_s != jnp.int32(_SENT)
        idx_s = lax.shift_right_arithmetic(comp_s, 14)
        ival_s = comp_s & jnp.int32(16383)
        nxt = _lane_shift_up(idx_s, iota)
        is_last = valid & ((nxt != idx_s) | (iota == L - 1))
        rel_s = jnp.where(valid, idx_s - r0, 0)
        plsc.store_scatter(table_v, [rel_s], ival_s, mask=is_last)
        return carry

    lax.fori_loop(0, B // L, scan_body, 0)

    def comp_body(v, cnt_vec):
        w = table_v[pl.ds(v * L, L)]
        mask = w >= 0
        m_vec = r0 + v * L + iota
        inc = jnp.where(mask, jnp.int32(1), jnp.int32(0))
        pos = cnt_vec + plsc.cumsum(inc) - 1
        row = lax.shift_right_logical(pos, 6)
        col = pos & jnp.int32(CH - 1)
        plsc.store_scatter(mlist_v, [row, col], m_vec, mask=mask)
        plsc.store_scatter(wlist_v, [row, col], w, mask=mask)
        return cnt_vec + plsc.all_reduce_population_count(mask)

    cnt_vec = lax.fori_loop(0, R // L, comp_body, jnp.zeros((L,), jnp.int32))
    cnt = cnt_vec[0]
    nch = (cnt + CH - 1) // CH
    pad_end = nch * CH

    m0 = mlist_v[0, pl.ds(0, L)][0]
    w0 = wlist_v[0, pl.ds(0, L)][0]

    def pad_body(p, carry):
        pos = cnt + p * L + iota
        maskp = pos < pad_end
        row = lax.shift_right_logical(pos, 6)
        col = pos & jnp.int32(CH - 1)
        plsc.store_scatter(mlist_v, [row, col], jnp.full((L,), 1, jnp.int32) * m0,
                           mask=maskp)
        plsc.store_scatter(wlist_v, [row, col], jnp.full((L,), 1, jnp.int32) * w0,
                           mask=maskp)
        return carry

    lax.fori_loop(0, CH // L, pad_body, 0)

    def chunk_body(k, carry):
        widx = wlist_v.at[k]
        midx = mlist_v.at[k]
        g0 = pltpu.async_copy(
            nope_hbm.at[:, 0, pl.ds(0, 128)].at[widx], pbufa_v, sem_a)
        gr = pltpu.async_copy(rope_hbm.at[:, 0, pl.ds(0, 128)].at[widx],
                              pbufc_v, sem_c)
        g0.wait()
        for j in range(1, 4):
            buf_prev = pbufa_v if (j - 1) % 2 == 0 else pbufb_v
            buf_cur = pbufb_v if (j - 1) % 2 == 0 else pbufa_v
            g = pltpu.async_copy(
                nope_hbm.at[:, 0, pl.ds(128 * j, 128)].at[widx], buf_cur, sem_b)
            s = pltpu.async_copy(
                buf_prev, out_hbm.at[:, 0, pl.ds(128 * (j - 1), 128)].at[midx],
                sem_a)
            g.wait()
            s.wait()
        buf_last = pbufb_v if 3 % 2 == 1 else pbufa_v
        s3 = pltpu.async_copy(
            buf_last, out_hbm.at[:, 0, pl.ds(128 * 3, 128)].at[midx], sem_b)
        mvecs = [mlist_v[k, pl.ds(g * L, L)] for g in range(CH // L)]
        s3.wait()
        gr.wait()
        cbufs = [cbufa_v, cbufb_v, cbufc_v, cbufd_v]
        for i in range(CH):
            half = cbufs[i // CP]
            for c in range(ROPE // L):
                half[i % CP, 0, pl.ds(NOPE + c * L, L)] = \
                    pbufc_v[i, pl.ds(c * L, L)]
        descs = []
        for i in range(CH):
            half = cbufs[i // CP]
            m = mvecs[i // L][i % L]
            descs.append(
                pltpu.async_copy(half.at[pl.ds(i % CP, 1), 0, pl.ds(NOPE, ROPE)],
                                 out_hbm.at[pl.ds(m, 1), 0, pl.ds(NOPE, ROPE)],
                                 sem_r))
        for desc in descs:
            desc.wait()
        return carry

    lax.fori_loop(0, nch, chunk_body, 0)


@functools.partial(
    pl.kernel,
    out_type=(),
    mesh=plsc.VectorSubcoreMesh(core_axis_name="c", subcore_axis_name="s"),
    compiler_params=pltpu.CompilerParams(
        needs_layout_passes=False, use_tc_tiling_on_sc=True),
    scratch_types=[
        pltpu.VMEM((B,), jnp.int32),
        pltpu.VMEM((R,), jnp.int32),
        pltpu.VMEM((NCHMAX, CH), jnp.int32),
        pltpu.VMEM((NCHMAX, CH), jnp.int32),
        pltpu.VMEM((CP, 1, D), jnp.float32),
        pltpu.VMEM((CP, 1, D), jnp.float32),
        pltpu.VMEM((CP, 1, D), jnp.float32),
        pltpu.VMEM((CP, 1, D), jnp.float32),
        pltpu.VMEM((CH, 128), jnp.float32),
        pltpu.VMEM((CH, 128), jnp.float32),
        pltpu.VMEM((CH, 128), jnp.float32),
        pltpu.SemaphoreType.DMA,
        pltpu.SemaphoreType.DMA,
        pltpu.SemaphoreType.DMA,
        pltpu.SemaphoreType.DMA,
        pltpu.SemaphoreType.DMA,
    ],
)
def _sc_scatter(loc_hbm, nope_hbm, rope_hbm, out_hbm, *rest):
    _sc_body(loc_hbm, nope_hbm, rope_hbm, out_hbm, *rest)


def kernel(kv_buffer, loc, cache_k_nope, cache_k_rope):
    loc32 = loc.astype(jnp.int32)
    rope_p = jnp.concatenate(
        [cache_k_rope, jnp.zeros((B, 1, 128 - ROPE), jnp.float32)], axis=-1)
    out_ref = jax.new_ref(kv_buffer)
    _sc_scatter(loc32, cache_k_nope, rope_p, out_ref)
    return out_ref[...]
